# baseline (speedup 1.0000x reference)
import jax
import jax.numpy as jnp
from jax import lax
from jax.experimental import pallas as pl
from jax.experimental.pallas import tpu as pltpu
from jax.experimental.pallas import tpu_sc as plsc

NUM_USERS = 100000
EMBED = 128
N_NODES = 10000
DEG = 32

NC, NS = 2, 16
NW = NC * NS

N_SLICES = 1
NS_NODES = N_NODES // N_SLICES

E_ROWS = NS_NODES * DEG
E_PER_W = E_ROWS // NW
CHUNK = 200
N_CHUNKS = E_PER_W // CHUNK
NBUF = 4

U_PAD = 10240
U_PER_W = U_PAD // NW


def _sc_gather_body(neigh_hbm, node_hbm, table_hbm,
                    e_out_hbm, u_out_hbm,
                    idx_all, rows2, gsem, wsem):
    wid = lax.axis_index("s") * NC + lax.axis_index("c")
    ebase = wid * E_PER_W

    pltpu.sync_copy(neigh_hbm.at[pl.ds(ebase, E_PER_W)], idx_all)

    def start_gather(i, b):
        pltpu.async_copy(
            table_hbm.at[idx_all.at[pl.ds(i * CHUNK, CHUNK)]],
            rows2.at[b], gsem.at[b])

    def gather_wait(b):
        pltpu.make_async_copy(
            table_hbm.at[idx_all.at[pl.ds(0, CHUNK)]],
            rows2.at[b], gsem.at[b]).wait()

    def start_write(i, b):
        pltpu.async_copy(rows2.at[b],
                         e_out_hbm.at[pl.ds(ebase + i * CHUNK, CHUNK)],
                         wsem.at[b])

    def write_wait(b):
        pltpu.make_async_copy(
            rows2.at[b], e_out_hbm.at[pl.ds(ebase, CHUNK)],
            wsem.at[b]).wait()

    start_gather(0, 0)
    start_gather(1, 1)

    def chunk(i, _):
        b = lax.rem(i, NBUF)
        gather_wait(b)
        start_write(i, b)

        @pl.when(i + 2 < N_CHUNKS)
        def _():
            nb = lax.rem(i + 2, NBUF)
            @pl.when(i >= 2)
            def _():
                write_wait(nb)
            start_gather(i + 2, nb)

        return 0

    lax.fori_loop(0, N_CHUNKS, chunk, 0)
    write_wait(lax.rem(N_CHUNKS - 2, NBUF))
    write_wait(lax.rem(N_CHUNKS - 1, NBUF))

    ubase = wid * U_PER_W
    idx_u = idx_all.at[pl.ds(0, U_PER_W)]
    rows_u = rows2.at[0].at[pl.ds(0, U_PER_W)]
    pltpu.sync_copy(node_hbm.at[pl.ds(ubase, U_PER_W)], idx_u)
    pltpu.async_copy(table_hbm.at[idx_u], rows_u, gsem.at[0]).wait()
    pltpu.sync_copy(rows_u, u_out_hbm.at[pl.ds(ubase, U_PER_W)])


@jax.jit
def _sc_gather(neigh_flat, nodes_pad, table):
    mesh = plsc.VectorSubcoreMesh(core_axis_name="c", subcore_axis_name="s")
    return pl.kernel(
        _sc_gather_body,
        out_type=(
            jax.ShapeDtypeStruct((E_ROWS, EMBED), jnp.float32),
            jax.ShapeDtypeStruct((U_PAD, EMBED), jnp.float32),
        ),
        mesh=mesh,
        scratch_types=[
            pltpu.VMEM((E_PER_W,), jnp.int32),
            pltpu.VMEM((NBUF, CHUNK, EMBED), jnp.float32),
            pltpu.SemaphoreType.DMA((NBUF,)),
            pltpu.SemaphoreType.DMA((NBUF,)),
        ],
    )(neigh_flat, nodes_pad, table)


T = 400


def _tc_mlp_body(ea_ref, eb_ref, u_ref, w1a_ref, w1b_ref, b1_ref, w2_ref,
                 b2_ref, w3b_ref, out_ref):
    e3 = jnp.concatenate([ea_ref[...], eb_ref[...]], axis=0)
    e2 = e3.reshape(DEG * T, EMBED).astype(jnp.bfloat16)
    u1 = jnp.dot(u_ref[...], w1b_ref[...],
                 preferred_element_type=jnp.float32) + b1_ref[...]
    x = jnp.dot(e2, w1a_ref[...], preferred_element_type=jnp.float32)
    x = jnp.maximum(x.reshape(DEG, T, EMBED) + u1[None, :, :], 0.0)
    xb = x.reshape(DEG * T, EMBED).astype(jnp.bfloat16)
    x2 = jnp.dot(xb, w2_ref[...], preferred_element_type=jnp.float32)
    x2 = jnp.maximum(x2 + b2_ref[...], 0.0).astype(jnp.bfloat16)
    p = jnp.exp(jnp.dot(x2, w3b_ref[...],
                        preferred_element_type=jnp.float32))
    p3 = p.reshape(DEG, T, EMBED)
    den = jnp.sum(p3, axis=0)
    num = jnp.sum(p3 * e3, axis=0)
    out_ref[...] = num / den


@jax.jit
def _tc_mlp(e3, u_rep, w1a, w1b, b1, w2, b2, w3b):
    grid = (NS_NODES // T,)
    full = lambda shape: pl.BlockSpec(shape, lambda i: (0,) * len(shape))
    return pl.pallas_call(
        _tc_mlp_body,
        grid=grid,
        in_specs=[
            pl.BlockSpec((DEG // 2, T, EMBED), lambda i: (0, i, 0)),
            pl.BlockSpec((DEG // 2, T, EMBED), lambda i: (1, i, 0)),
            pl.BlockSpec((T, EMBED), lambda i: (i, 0)),
            full((EMBED, EMBED)),
            full((EMBED, EMBED)),
            full((1, EMBED)),
            full((EMBED, EMBED)),
            full((1, EMBED)),
            full((EMBED, EMBED)),
        ],
        out_specs=pl.BlockSpec((T, EMBED), lambda i: (i, 0)),
        out_shape=jax.ShapeDtypeStruct((NS_NODES, EMBED), jnp.float32),
        compiler_params=pltpu.CompilerParams(
            dimension_semantics=("parallel",)),
    )(e3, e3, u_rep, w1a, w1b, b1, w2, b2, w3b)


def kernel(nodes, to_neighs, u2e_weight, W1, b1, W2, b2, W3, b3):
    w1a = W1[:EMBED].astype(jnp.bfloat16)
    w1b = W1[EMBED:]
    b1r = b1.reshape(1, EMBED)
    w2 = W2.astype(jnp.bfloat16)
    b2r = b2.reshape(1, EMBED)
    w3b = jnp.broadcast_to(W3, (EMBED, EMBED)).astype(jnp.bfloat16)

    outs = []
    for s in range(N_SLICES):
        lo = s * NS_NODES
        neigh_flat = to_neighs[lo:lo + NS_NODES].T.reshape(E_ROWS)
        nodes_pad = jnp.concatenate(
            [nodes[lo:lo + NS_NODES],
             jnp.zeros((U_PAD - NS_NODES,), jnp.int32)])
        e_gath, u_gath = _sc_gather(neigh_flat, nodes_pad, u2e_weight)
        outs.append(_tc_mlp(
            e_gath.reshape(DEG, NS_NODES, EMBED),
            u_gath,
            w1a, w1b, b1r, w2, b2r, w3b,
        ))
    return jnp.concatenate(outs, axis=0)


# speedup vs baseline: 1.0007x; 1.0007x over previous
import jax
import jax.numpy as jnp
from jax import lax
from jax.experimental import pallas as pl
from jax.experimental.pallas import tpu as pltpu
from jax.experimental.pallas import tpu_sc as plsc

NUM_USERS = 100000
EMBED = 128
N_NODES = 10000
DEG = 32

NC, NS = 2, 16
NW = NC * NS

N_SLICES = 1
NS_NODES = N_NODES // N_SLICES

E_ROWS = NS_NODES * DEG
E_PER_W = E_ROWS // NW
CHUNK = 200
N_CHUNKS = E_PER_W // CHUNK
NBUF = 4

U_PAD = 10240
U_PER_W = U_PAD // NW


def _sc_gather_body(neigh_hbm, node_hbm, table_hbm,
                    e_out_hbm, u_out_hbm,
                    idx_all, rows2, gsem, wsem):
    wid = lax.axis_index("s") * NC + lax.axis_index("c")
    ebase = wid * E_PER_W

    pltpu.sync_copy(neigh_hbm.at[pl.ds(ebase, E_PER_W)], idx_all)

    def start_gather(i, b):
        pltpu.async_copy(
            table_hbm.at[idx_all.at[pl.ds(i * CHUNK, CHUNK)]],
            rows2.at[b], gsem.at[b])

    def gather_wait(b):
        pltpu.make_async_copy(
            table_hbm.at[idx_all.at[pl.ds(0, CHUNK)]],
            rows2.at[b], gsem.at[b]).wait()

    def start_write(i, b):
        pltpu.async_copy(rows2.at[b],
                         e_out_hbm.at[pl.ds(ebase + i * CHUNK, CHUNK)],
                         wsem.at[b])

    def write_wait(b):
        pltpu.make_async_copy(
            rows2.at[b], e_out_hbm.at[pl.ds(ebase, CHUNK)],
            wsem.at[b]).wait()

    start_gather(0, 0)
    start_gather(1, 1)

    def chunk(i, _):
        b = lax.rem(i, NBUF)
        gather_wait(b)
        start_write(i, b)

        @pl.when(i + 2 < N_CHUNKS)
        def _():
            nb = lax.rem(i + 2, NBUF)
            @pl.when(i >= 2)
            def _():
                write_wait(nb)
            start_gather(i + 2, nb)

        return 0

    lax.fori_loop(0, N_CHUNKS, chunk, 0)
    write_wait(lax.rem(N_CHUNKS - 2, NBUF))
    write_wait(lax.rem(N_CHUNKS - 1, NBUF))

    ubase = wid * U_PER_W
    idx_u = idx_all.at[pl.ds(0, U_PER_W)]
    rows_u = rows2.at[0].at[pl.ds(0, U_PER_W)]
    pltpu.sync_copy(node_hbm.at[pl.ds(ubase, U_PER_W)], idx_u)
    pltpu.async_copy(table_hbm.at[idx_u], rows_u, gsem.at[0]).wait()
    pltpu.sync_copy(rows_u, u_out_hbm.at[pl.ds(ubase, U_PER_W)])


@jax.jit
def _sc_gather(neigh_flat, nodes_pad, table):
    mesh = plsc.VectorSubcoreMesh(core_axis_name="c", subcore_axis_name="s")
    return pl.kernel(
        _sc_gather_body,
        out_type=(
            jax.ShapeDtypeStruct((E_ROWS, EMBED), jnp.float32),
            jax.ShapeDtypeStruct((U_PAD, EMBED), jnp.float32),
        ),
        mesh=mesh,
        scratch_types=[
            pltpu.VMEM((E_PER_W,), jnp.int32),
            pltpu.VMEM((NBUF, CHUNK, EMBED), jnp.float32),
            pltpu.SemaphoreType.DMA((NBUF,)),
            pltpu.SemaphoreType.DMA((NBUF,)),
        ],
    )(neigh_flat, nodes_pad, table)


T = 400


def _tc_mlp_body(e_ref, u_ref, w1a_ref, w1b_ref, b1_ref, w2_ref, b2_ref,
                 w3b_ref, out_ref):
    e3 = e_ref[...]
    e2 = e3.reshape(DEG * T, EMBED).astype(jnp.bfloat16)
    u1 = jnp.dot(u_ref[...], w1b_ref[...],
                 preferred_element_type=jnp.float32) + b1_ref[...]
    x = jnp.dot(e2, w1a_ref[...], preferred_element_type=jnp.float32)
    x = jnp.maximum(x.reshape(DEG, T, EMBED) + u1[None, :, :], 0.0)
    xb = x.reshape(DEG * T, EMBED).astype(jnp.bfloat16)
    x2 = jnp.dot(xb, w2_ref[...], preferred_element_type=jnp.float32)
    x2 = jnp.maximum(x2 + b2_ref[...], 0.0).astype(jnp.bfloat16)
    p = jnp.exp(jnp.dot(x2, w3b_ref[...],
                        preferred_element_type=jnp.float32))
    p3 = p.reshape(DEG, T, EMBED)
    den = jnp.sum(p3, axis=0)
    num = jnp.sum(p3 * e3, axis=0)
    out_ref[...] = num / den


@jax.jit
def _tc_mlp(e3, u_rep, w1a, w1b, b1, w2, b2, w3b):
    grid = (NS_NODES // T,)
    full = lambda shape: pl.BlockSpec(shape, lambda i: (0,) * len(shape))
    return pl.pallas_call(
        _tc_mlp_body,
        grid=grid,
        in_specs=[
            pl.BlockSpec((DEG, T, EMBED), lambda i: (0, i, 0)),
            pl.BlockSpec((T, EMBED), lambda i: (i, 0)),
            full((EMBED, EMBED)),
            full((EMBED, EMBED)),
            full((1, EMBED)),
            full((EMBED, EMBED)),
            full((1, EMBED)),
            full((EMBED, EMBED)),
        ],
        out_specs=pl.BlockSpec((T, EMBED), lambda i: (i, 0)),
        out_shape=jax.ShapeDtypeStruct((NS_NODES, EMBED), jnp.float32),
        compiler_params=pltpu.CompilerParams(
            dimension_semantics=("parallel",)),
    )(e3, u_rep, w1a, w1b, b1, w2, b2, w3b)


def kernel(nodes, to_neighs, u2e_weight, W1, b1, W2, b2, W3, b3):
    w1a = W1[:EMBED].astype(jnp.bfloat16)
    w1b = W1[EMBED:]
    b1r = b1.reshape(1, EMBED)
    w2 = W2.astype(jnp.bfloat16)
    b2r = b2.reshape(1, EMBED)
    w3b = jnp.broadcast_to(W3, (EMBED, EMBED)).astype(jnp.bfloat16)

    outs = []
    for s in range(N_SLICES):
        lo = s * NS_NODES
        neigh_flat = to_neighs[lo:lo + NS_NODES].T.reshape(E_ROWS)
        nodes_pad = jnp.concatenate(
            [nodes[lo:lo + NS_NODES],
             jnp.zeros((U_PAD - NS_NODES,), jnp.int32)])
        e_gath, u_gath = _sc_gather(neigh_flat, nodes_pad, u2e_weight)
        outs.append(_tc_mlp(
            e_gath.reshape(DEG, NS_NODES, EMBED),
            u_gath,
            w1a, w1b, b1r, w2, b2r, w3b,
        ))
    return jnp.concatenate(outs, axis=0)
